# baseline (speedup 1.0000x reference)
import jax
import jax.numpy as jnp
from jax import lax
from jax.experimental import pallas as pl
from jax.experimental.pallas import tpu as pltpu
from jax.experimental.pallas import tpu_sc as plsc

N = 320000
D = 128
K = 1000
KP = 1024

NUM_CORES = 2
NUM_SUBCORES = 16
NW = NUM_CORES * NUM_SUBCORES

BT = 512
M_TC = 225 * BT
SC_ROWS = N - M_TC
ROWS_PER_TILE = SC_ROWS // NW
SCATTER_B = 100
CHUNK = 2 * SCATTER_B
CHUNKS_PER_TILE = ROWS_PER_TILE // CHUNK
TROWS = N // SCATTER_B


def _sc_body(feat_hbm, tgt2_hbm, tgt1_hbm, zsum_hbm, zc_hbm,
             sums_out, cnt_out,
             fbuf0, fbuf1, tb00, tb01, tb10, tb11, tv, cnt_local, acc_sh,
             sem_in0, sem_in1, sem_s0, sem_s1):
    c = lax.axis_index("c")
    s = lax.axis_index("s")
    wid = c * NUM_SUBCORES + s

    fbufs = (fbuf0, fbuf1)
    tbs = ((tb00, tb01), (tb10, tb11))
    sem_in = (sem_in0, sem_in1)
    sem_s = (sem_s0, sem_s1)

    zrows = KP // NUM_SUBCORES
    pltpu.sync_copy(zsum_hbm.at[pl.ds(s * zrows, zrows)],
                    acc_sh.at[pl.ds(s * zrows, zrows)])
    pltpu.sync_copy(zc_hbm, cnt_local)
    pltpu.sync_copy(
        tgt1_hbm.at[pl.ds(M_TC + wid * ROWS_PER_TILE, ROWS_PER_TILE)], tv)
    plsc.subcore_barrier()

    base_f = M_TC + wid * ROWS_PER_TILE
    base_t = (M_TC + wid * ROWS_PER_TILE) // SCATTER_B

    def start_in(i, b):
        pltpu.async_copy(feat_hbm.at[pl.ds(base_f + i * CHUNK, CHUNK)],
                         fbufs[b], sem_in[b])
        pltpu.async_copy(tgt2_hbm.at[pl.ds(base_t + i * 2, 1)],
                         tbs[b][0], sem_in[b])
        pltpu.async_copy(tgt2_hbm.at[pl.ds(base_t + i * 2 + 1, 1)],
                         tbs[b][1], sem_in[b])

    def wait_in(i, b):
        pltpu.make_async_copy(feat_hbm.at[pl.ds(base_f + i * CHUNK, CHUNK)],
                              fbufs[b], sem_in[b]).wait()
        pltpu.make_async_copy(tgt2_hbm.at[pl.ds(base_t + i * 2, 1)],
                              tbs[b][0], sem_in[b]).wait()
        pltpu.make_async_copy(tgt2_hbm.at[pl.ds(base_t + i * 2 + 1, 1)],
                              tbs[b][1], sem_in[b]).wait()

    def start_scatter(b):
        for h in range(2):
            pltpu.async_copy(fbufs[b].at[pl.ds(h * SCATTER_B, SCATTER_B)],
                             acc_sh.at[tbs[b][h].at[0]], sem_s[b], add=True)

    def wait_scatter(b):
        for h in range(2):
            pltpu.make_async_copy(fbufs[b].at[pl.ds(h * SCATTER_B, SCATTER_B)],
                                  acc_sh.at[tbs[b][h].at[0]], sem_s[b]).wait()

    start_in(0, 0)

    def pair(j, carry):
        i0 = 2 * j
        wait_in(i0, 0)
        start_scatter(0)

        @pl.when(j > 0)
        def _():
            wait_scatter(1)
        start_in(i0 + 1, 1)
        wait_in(i0 + 1, 1)
        start_scatter(1)

        @pl.when(j < CHUNKS_PER_TILE // 2 - 1)
        def _():
            wait_scatter(0)
            start_in(i0 + 2, 0)
        return carry

    lax.fori_loop(0, CHUNKS_PER_TILE // 2, pair, 0)

    iota16 = lax.iota(jnp.int32, 16)
    ones16 = jnp.ones((16,), jnp.float32)

    def cbody(i, carry):
        t16 = tv[pl.ds(i * 16, 16)]
        idx = ((t16 >> 7) << 11) + (iota16 << 7) + (t16 & 127)
        plsc.addupdate_scatter(cnt_local, [idx], ones16)
        return carry

    lax.fori_loop(0, ROWS_PER_TILE // 16, cbody, 0)
    pltpu.sync_copy(cnt_local, cnt_out.at[wid])

    wait_scatter(0)
    wait_scatter(1)
    plsc.subcore_barrier()

    @pl.when(s == 0)
    def _():
        pltpu.sync_copy(acc_sh, sums_out.at[c])


_sc_segment_sums = pl.kernel(
    _sc_body,
    out_type=(
        jax.ShapeDtypeStruct((NUM_CORES, KP, D), jnp.float32),
        jax.ShapeDtypeStruct((NW, 128 * 128), jnp.float32),
    ),
    mesh=plsc.VectorSubcoreMesh(core_axis_name="c", subcore_axis_name="s"),
    compiler_params=pltpu.CompilerParams(use_tc_tiling_on_sc=False,
                                         needs_layout_passes=False),
    scratch_types=[
        pltpu.VMEM((CHUNK, D), jnp.float32),
        pltpu.VMEM((CHUNK, D), jnp.float32),
        pltpu.VMEM((1, SCATTER_B), jnp.int32),
        pltpu.VMEM((1, SCATTER_B), jnp.int32),
        pltpu.VMEM((1, SCATTER_B), jnp.int32),
        pltpu.VMEM((1, SCATTER_B), jnp.int32),
        pltpu.VMEM((ROWS_PER_TILE,), jnp.int32),
        pltpu.VMEM((128 * 128,), jnp.float32),
        pltpu.VMEM_SHARED((KP, D), jnp.float32),
        pltpu.SemaphoreType.DMA,
        pltpu.SemaphoreType.DMA,
        pltpu.SemaphoreType.DMA,
        pltpu.SemaphoreType.DMA,
    ],
)


def _tc_partial_body(t_ref, f_ref, p_ref):
    i = pl.program_id(0)

    @pl.when(i == 0)
    def _():
        p_ref[...] = jnp.zeros_like(p_ref)

    t = t_ref[0, 0, :]
    oh = (lax.broadcasted_iota(jnp.int32, (BT, KP), 1)
          == t[:, None]).astype(jnp.bfloat16)
    fb = f_ref[...].astype(jnp.bfloat16)
    f1 = jnp.concatenate(
        [fb, jnp.ones((BT, 1), jnp.bfloat16)], axis=1)
    p_ref[...] += lax.dot_general(
        oh, f1, (((0,), (0,)), ((), ())),
        preferred_element_type=jnp.float32)


_tc_partial = pl.pallas_call(
    _tc_partial_body,
    grid=(M_TC // BT,),
    in_specs=[
        pl.BlockSpec((1, 1, BT), lambda i: (i, 0, 0)),
        pl.BlockSpec((BT, D), lambda i: (i, 0)),
    ],
    out_specs=pl.BlockSpec((KP, D + 1), lambda i: (0, 0)),
    out_shape=jax.ShapeDtypeStruct((KP, D + 1), jnp.float32),
)


def _tc_body(sums_ref, cnt_ref, p_ref, fma_ref, mu_ref):
    s = sums_ref[0] + sums_ref[1] + p_ref[:, :D]
    cnt_a = jnp.sum(cnt_ref[...], axis=0)
    b = jnp.sum(cnt_a.reshape(8, 16, 128), axis=1)
    kk = lax.broadcasted_iota(jnp.int32, (KP, 1), 0)
    sel = (lax.broadcasted_iota(jnp.int32, (KP, 8), 1) == (kk >> 7))
    c1 = jnp.dot(sel.astype(jnp.float32), b,
                 preferred_element_type=jnp.float32)
    m_iota = lax.broadcasted_iota(jnp.int32, (KP, 128), 1)
    pick = (m_iota == (kk & 127)).astype(jnp.float32)
    cnt = jnp.sum(c1 * pick, axis=1, keepdims=True) + p_ref[:, D:]

    mu = jnp.sum(s, axis=0, keepdims=True) / float(N)
    has = cnt > 0.0
    fm = jnp.where(has, s / jnp.where(has, cnt, 1.0) - mu, mu)
    fm_t = fm.T[:, :K]
    norm = jnp.sqrt(jnp.sum(fm_t * fm_t))
    fma_ref[...] = fm_t / norm
    mu_ref[...] = mu


_tc_epilogue = pl.pallas_call(
    _tc_body,
    out_shape=(
        jax.ShapeDtypeStruct((D, K), jnp.float32),
        jax.ShapeDtypeStruct((1, D), jnp.float32),
    ),
)


@jax.jit
def kernel(features, targets):
    t2 = targets.reshape(TROWS, SCATTER_B)
    zsum = jnp.zeros((KP, D), jnp.float32)
    zc = jnp.zeros((128 * 128,), jnp.float32)
    sums, cnt = _sc_segment_sums(features, t2, targets, zsum, zc)
    t3 = targets[:M_TC].reshape(M_TC // BT, 1, BT)
    p = _tc_partial(t3, features[:M_TC])
    fma, mu = _tc_epilogue(sums, cnt.reshape(NW, 128, 128), p)
    return fma, mu.reshape(D)


# speedup vs baseline: 1.5834x; 1.5834x over previous
import jax
import jax.numpy as jnp
from jax import lax
from jax.experimental import pallas as pl
from jax.experimental.pallas import tpu as pltpu
from jax.experimental.pallas import tpu_sc as plsc

N = 320000
D = 128
K = 1000
KP = 1024

NUM_CORES = 2
NUM_SUBCORES = 16
NW = NUM_CORES * NUM_SUBCORES

BT = 1024
M_TC = 125 * BT
SC_ROWS = N - M_TC
ROWS_PER_TILE = SC_ROWS // NW
CNT_ROWS_PER_TILE = N // NW
SCATTER_B = 100
CHUNK = 2 * SCATTER_B
CHUNKS_PER_TILE = ROWS_PER_TILE // CHUNK
TROWS = N // SCATTER_B


def _sc_body(feat_hbm, tgt2_hbm, tgt1_hbm, zsum_hbm, zc_hbm,
             sums_out, cnt_out,
             fbuf0, fbuf1, tb00, tb01, tb10, tb11, tv, cnt_local, acc_sh,
             sem_in0, sem_in1, sem_s0, sem_s1):
    c = lax.axis_index("c")
    s = lax.axis_index("s")
    wid = c * NUM_SUBCORES + s

    fbufs = (fbuf0, fbuf1)
    tbs = ((tb00, tb01), (tb10, tb11))
    sem_in = (sem_in0, sem_in1)
    sem_s = (sem_s0, sem_s1)

    zrows = KP // NUM_SUBCORES
    pltpu.sync_copy(zsum_hbm.at[pl.ds(s * zrows, zrows)],
                    acc_sh.at[pl.ds(s * zrows, zrows)])
    pltpu.sync_copy(zc_hbm, cnt_local)
    pltpu.sync_copy(
        tgt1_hbm.at[pl.ds(wid * CNT_ROWS_PER_TILE, CNT_ROWS_PER_TILE)], tv)
    plsc.subcore_barrier()

    base_f = M_TC + wid * ROWS_PER_TILE
    base_t = (M_TC + wid * ROWS_PER_TILE) // SCATTER_B

    def start_in(i, b):
        pltpu.async_copy(feat_hbm.at[pl.ds(base_f + i * CHUNK, CHUNK)],
                         fbufs[b], sem_in[b])
        pltpu.async_copy(tgt2_hbm.at[pl.ds(base_t + i * 2, 1)],
                         tbs[b][0], sem_in[b])
        pltpu.async_copy(tgt2_hbm.at[pl.ds(base_t + i * 2 + 1, 1)],
                         tbs[b][1], sem_in[b])

    def wait_in(i, b):
        pltpu.make_async_copy(feat_hbm.at[pl.ds(base_f + i * CHUNK, CHUNK)],
                              fbufs[b], sem_in[b]).wait()
        pltpu.make_async_copy(tgt2_hbm.at[pl.ds(base_t + i * 2, 1)],
                              tbs[b][0], sem_in[b]).wait()
        pltpu.make_async_copy(tgt2_hbm.at[pl.ds(base_t + i * 2 + 1, 1)],
                              tbs[b][1], sem_in[b]).wait()

    def start_scatter(b):
        for h in range(2):
            pltpu.async_copy(fbufs[b].at[pl.ds(h * SCATTER_B, SCATTER_B)],
                             acc_sh.at[tbs[b][h].at[0]], sem_s[b], add=True)

    def wait_scatter(b):
        for h in range(2):
            pltpu.make_async_copy(fbufs[b].at[pl.ds(h * SCATTER_B, SCATTER_B)],
                                  acc_sh.at[tbs[b][h].at[0]], sem_s[b]).wait()

    start_in(0, 0)

    def pair(j, carry):
        i0 = 2 * j
        wait_in(i0, 0)
        start_scatter(0)

        @pl.when(j > 0)
        def _():
            wait_scatter(1)
        start_in(i0 + 1, 1)
        wait_in(i0 + 1, 1)
        start_scatter(1)

        @pl.when(j < CHUNKS_PER_TILE // 2 - 1)
        def _():
            wait_scatter(0)
            start_in(i0 + 2, 0)
        return carry

    lax.fori_loop(0, CHUNKS_PER_TILE // 2, pair, 0)

    iota16 = lax.iota(jnp.int32, 16)
    ones16 = jnp.ones((16,), jnp.float32)

    def cbody(i, carry):
        t16 = tv[pl.ds(i * 16, 16)]
        idx = ((t16 >> 7) << 11) + (iota16 << 7) + (t16 & 127)
        plsc.addupdate_scatter(cnt_local, [idx], ones16)
        return carry

    lax.fori_loop(0, CNT_ROWS_PER_TILE // 16, cbody, 0)
    pltpu.sync_copy(cnt_local, cnt_out.at[wid])

    wait_scatter(0)
    wait_scatter(1)
    plsc.subcore_barrier()

    @pl.when(s == 0)
    def _():
        pltpu.sync_copy(acc_sh, sums_out.at[c])


_sc_segment_sums = pl.kernel(
    _sc_body,
    out_type=(
        jax.ShapeDtypeStruct((NUM_CORES, KP, D), jnp.float32),
        jax.ShapeDtypeStruct((NW, 128 * 128), jnp.float32),
    ),
    mesh=plsc.VectorSubcoreMesh(core_axis_name="c", subcore_axis_name="s"),
    compiler_params=pltpu.CompilerParams(use_tc_tiling_on_sc=False,
                                         needs_layout_passes=False),
    scratch_types=[
        pltpu.VMEM((CHUNK, D), jnp.float32),
        pltpu.VMEM((CHUNK, D), jnp.float32),
        pltpu.VMEM((1, SCATTER_B), jnp.int32),
        pltpu.VMEM((1, SCATTER_B), jnp.int32),
        pltpu.VMEM((1, SCATTER_B), jnp.int32),
        pltpu.VMEM((1, SCATTER_B), jnp.int32),
        pltpu.VMEM((CNT_ROWS_PER_TILE,), jnp.int32),
        pltpu.VMEM((128 * 128,), jnp.float32),
        pltpu.VMEM_SHARED((KP, D), jnp.float32),
        pltpu.SemaphoreType.DMA,
        pltpu.SemaphoreType.DMA,
        pltpu.SemaphoreType.DMA,
        pltpu.SemaphoreType.DMA,
    ],
)


def _tc_partial_body(t_ref, f_ref, p_ref):
    i = pl.program_id(0)

    @pl.when(i == 0)
    def _():
        p_ref[...] = jnp.zeros_like(p_ref)

    halves = []
    for h in range(2):
        t = t_ref[h, 0, :]
        halves.append((lax.broadcasted_iota(jnp.int32, (512, KP), 1)
                       == t[:, None]).astype(jnp.bfloat16))
    oh = jnp.concatenate(halves, axis=0)
    fb = f_ref[...].astype(jnp.bfloat16)
    p_ref[...] += lax.dot_general(
        oh, fb, (((0,), (0,)), ((), ())),
        preferred_element_type=jnp.float32)


_tc_partial = pl.pallas_call(
    _tc_partial_body,
    grid=(M_TC // BT,),
    in_specs=[
        pl.BlockSpec((2, 1, 512), lambda i: (i, 0, 0)),
        pl.BlockSpec((BT, D), lambda i: (i, 0)),
    ],
    out_specs=pl.BlockSpec((KP, D), lambda i: (0, 0)),
    out_shape=jax.ShapeDtypeStruct((KP, D), jnp.float32),
)


def _tc_body(sums_ref, cnt_ref, p_ref, fma_ref, mu_ref):
    s = sums_ref[0] + sums_ref[1] + p_ref[...]
    cnt_a = jnp.sum(cnt_ref[...], axis=0)
    b = jnp.sum(cnt_a.reshape(8, 16, 128), axis=1)
    kk = lax.broadcasted_iota(jnp.int32, (KP, 1), 0)
    sel = (lax.broadcasted_iota(jnp.int32, (KP, 8), 1) == (kk >> 7))
    c1 = jnp.dot(sel.astype(jnp.float32), b,
                 preferred_element_type=jnp.float32)
    m_iota = lax.broadcasted_iota(jnp.int32, (KP, 128), 1)
    pick = (m_iota == (kk & 127)).astype(jnp.float32)
    cnt = jnp.sum(c1 * pick, axis=1, keepdims=True)

    mu = jnp.sum(s, axis=0, keepdims=True) / float(N)
    has = cnt > 0.0
    fm = jnp.where(has, s / jnp.where(has, cnt, 1.0) - mu, mu)
    fm_t = fm.T[:, :K]
    norm = jnp.sqrt(jnp.sum(fm_t * fm_t))
    fma_ref[...] = fm_t / norm
    mu_ref[...] = mu


_tc_epilogue = pl.pallas_call(
    _tc_body,
    out_shape=(
        jax.ShapeDtypeStruct((D, K), jnp.float32),
        jax.ShapeDtypeStruct((1, D), jnp.float32),
    ),
)


@jax.jit
def kernel(features, targets):
    t2 = targets.reshape(TROWS, SCATTER_B)
    zsum = jnp.zeros((KP, D), jnp.float32)
    zc = jnp.zeros((128 * 128,), jnp.float32)
    sums, cnt = _sc_segment_sums(features, t2, targets, zsum, zc)
    t3 = targets.reshape(N // 512, 1, 512)
    p = _tc_partial(t3, features)
    fma, mu = _tc_epilogue(sums, cnt.reshape(NW, 128, 128), p)
    return fma, mu.reshape(D)


# speedup vs baseline: 1.8255x; 1.1529x over previous
import jax
import jax.numpy as jnp
from jax import lax
from jax.experimental import pallas as pl
from jax.experimental.pallas import tpu as pltpu
from jax.experimental.pallas import tpu_sc as plsc

N = 320000
D = 128
K = 1000
KP = 1024

NUM_CORES = 2
NUM_SUBCORES = 16
NW = NUM_CORES * NUM_SUBCORES

BT = 1024
M_TC = 125 * BT
SC_ROWS = N - M_TC
ROWS_PER_TILE = SC_ROWS // NW
CNT_ROWS_PER_TILE = N // NW
SCATTER_B = 120
CHUNK = 2 * SCATTER_B
CHUNKS_PER_TILE = ROWS_PER_TILE // CHUNK


def _sc_body(feat_hbm, tgt1_hbm, zsum_hbm, zc_hbm,
             sums_out, cnt_out,
             fbuf0, fbuf1, tb00, tb01, tb10, tb11, tv, cnt_local, acc_sh,
             sem_in0, sem_in1, sem_s0, sem_s1):
    c = lax.axis_index("c")
    s = lax.axis_index("s")
    wid = c * NUM_SUBCORES + s

    fbufs = (fbuf0, fbuf1)
    tbs = ((tb00, tb01), (tb10, tb11))
    sem_in = (sem_in0, sem_in1)
    sem_s = (sem_s0, sem_s1)

    zrows = KP // NUM_SUBCORES
    pltpu.sync_copy(zsum_hbm.at[pl.ds(s * zrows, zrows)],
                    acc_sh.at[pl.ds(s * zrows, zrows)])
    pltpu.sync_copy(zc_hbm, cnt_local)
    pltpu.sync_copy(
        tgt1_hbm.at[pl.ds(wid * CNT_ROWS_PER_TILE, CNT_ROWS_PER_TILE)], tv)
    plsc.subcore_barrier()

    base_f = M_TC + wid * ROWS_PER_TILE

    def start_in(i, b):
        pltpu.async_copy(feat_hbm.at[pl.ds(base_f + i * CHUNK, CHUNK)],
                         fbufs[b], sem_in[b])
        pltpu.async_copy(tgt1_hbm.at[pl.ds(base_f + i * CHUNK, SCATTER_B)],
                         tbs[b][0], sem_in[b])
        pltpu.async_copy(
            tgt1_hbm.at[pl.ds(base_f + i * CHUNK + SCATTER_B, SCATTER_B)],
            tbs[b][1], sem_in[b])

    def wait_in(i, b):
        pltpu.make_async_copy(feat_hbm.at[pl.ds(base_f + i * CHUNK, CHUNK)],
                              fbufs[b], sem_in[b]).wait()
        pltpu.make_async_copy(tgt1_hbm.at[pl.ds(base_f + i * CHUNK, SCATTER_B)],
                              tbs[b][0], sem_in[b]).wait()
        pltpu.make_async_copy(
            tgt1_hbm.at[pl.ds(base_f + i * CHUNK + SCATTER_B, SCATTER_B)],
            tbs[b][1], sem_in[b]).wait()

    def start_scatter(b):
        for h in range(2):
            pltpu.async_copy(fbufs[b].at[pl.ds(h * SCATTER_B, SCATTER_B)],
                             acc_sh.at[tbs[b][h]], sem_s[b], add=True)

    def wait_scatter(b):
        for h in range(2):
            pltpu.make_async_copy(fbufs[b].at[pl.ds(h * SCATTER_B, SCATTER_B)],
                                  acc_sh.at[tbs[b][h]], sem_s[b]).wait()

    start_in(0, 0)

    def pair(j, carry):
        i0 = 2 * j
        wait_in(i0, 0)
        start_scatter(0)

        @pl.when(j > 0)
        def _():
            wait_scatter(1)
        start_in(i0 + 1, 1)
        wait_in(i0 + 1, 1)
        start_scatter(1)

        @pl.when(2 * j + 2 < CHUNKS_PER_TILE)
        def _():
            wait_scatter(0)
            start_in(i0 + 2, 0)
        return carry

    lax.fori_loop(0, CHUNKS_PER_TILE // 2, pair, 0)
    if CHUNKS_PER_TILE % 2:
        wait_in(CHUNKS_PER_TILE - 1, 0)
        start_scatter(0)

    iota16 = lax.iota(jnp.int32, 16)
    ones16 = jnp.ones((16,), jnp.float32)

    def cbody(i, carry):
        t16 = tv[pl.ds(i * 16, 16)]
        idx = ((t16 >> 7) << 11) + (iota16 << 7) + (t16 & 127)
        plsc.addupdate_scatter(cnt_local, [idx], ones16)
        return carry

    lax.fori_loop(0, CNT_ROWS_PER_TILE // 16, cbody, 0)
    pltpu.sync_copy(cnt_local, cnt_out.at[wid])

    wait_scatter(0)
    wait_scatter(1)
    plsc.subcore_barrier()

    @pl.when(s == 0)
    def _():
        pltpu.sync_copy(acc_sh, sums_out.at[c])


_sc_segment_sums = pl.kernel(
    _sc_body,
    out_type=(
        jax.ShapeDtypeStruct((NUM_CORES, KP, D), jnp.float32),
        jax.ShapeDtypeStruct((NW, 128 * 128), jnp.float32),
    ),
    mesh=plsc.VectorSubcoreMesh(core_axis_name="c", subcore_axis_name="s"),
    compiler_params=pltpu.CompilerParams(use_tc_tiling_on_sc=False,
                                         needs_layout_passes=False),
    scratch_types=[
        pltpu.VMEM((CHUNK, D), jnp.float32),
        pltpu.VMEM((CHUNK, D), jnp.float32),
        pltpu.VMEM((SCATTER_B,), jnp.int32),
        pltpu.VMEM((SCATTER_B,), jnp.int32),
        pltpu.VMEM((SCATTER_B,), jnp.int32),
        pltpu.VMEM((SCATTER_B,), jnp.int32),
        pltpu.VMEM((CNT_ROWS_PER_TILE,), jnp.int32),
        pltpu.VMEM((128 * 128,), jnp.float32),
        pltpu.VMEM_SHARED((KP, D), jnp.float32),
        pltpu.SemaphoreType.DMA,
        pltpu.SemaphoreType.DMA,
        pltpu.SemaphoreType.DMA,
        pltpu.SemaphoreType.DMA,
    ],
)


def _tc_partial_body(t_ref, f_ref, p_ref):
    i = pl.program_id(0)

    @pl.when(i == 0)
    def _():
        p_ref[...] = jnp.zeros_like(p_ref)

    fb = f_ref[...].astype(jnp.bfloat16)
    kio = lax.broadcasted_iota(jnp.int32, (KP, 512), 0)
    acc = None
    for h in range(2):
        t = t_ref[h]
        oh_t = (kio == t).astype(jnp.bfloat16)
        part = lax.dot_general(
            oh_t, fb[h * 512:(h + 1) * 512, :],
            (((1,), (0,)), ((), ())),
            preferred_element_type=jnp.float32)
        acc = part if acc is None else acc + part
    p_ref[...] += acc


_tc_partial = pl.pallas_call(
    _tc_partial_body,
    grid=(M_TC // BT,),
    in_specs=[
        pl.BlockSpec((2, 1, 512), lambda i: (i, 0, 0)),
        pl.BlockSpec((BT, D), lambda i: (i, 0)),
    ],
    out_specs=pl.BlockSpec((KP, D), lambda i: (0, 0)),
    out_shape=jax.ShapeDtypeStruct((KP, D), jnp.float32),
)


def _tc_body(sums_ref, cnt_ref, p_ref, fma_ref, mu_ref):
    s = sums_ref[0] + sums_ref[1] + p_ref[...]
    cnt_a = jnp.sum(cnt_ref[...], axis=0)
    b = jnp.sum(cnt_a.reshape(8, 16, 128), axis=1)
    kk = lax.broadcasted_iota(jnp.int32, (KP, 1), 0)
    sel = (lax.broadcasted_iota(jnp.int32, (KP, 8), 1) == (kk >> 7))
    c1 = jnp.dot(sel.astype(jnp.float32), b,
                 preferred_element_type=jnp.float32)
    m_iota = lax.broadcasted_iota(jnp.int32, (KP, 128), 1)
    pick = (m_iota == (kk & 127)).astype(jnp.float32)
    cnt = jnp.sum(c1 * pick, axis=1, keepdims=True)

    mu = jnp.sum(s, axis=0, keepdims=True) / float(N)
    has = cnt > 0.0
    fm = jnp.where(has, s / jnp.where(has, cnt, 1.0) - mu, mu)
    fm_t = fm.T[:, :K]
    norm = jnp.sqrt(jnp.sum(fm_t * fm_t))
    fma_ref[...] = fm_t / norm
    mu_ref[...] = mu


_tc_epilogue = pl.pallas_call(
    _tc_body,
    out_shape=(
        jax.ShapeDtypeStruct((D, K), jnp.float32),
        jax.ShapeDtypeStruct((1, D), jnp.float32),
    ),
)


@jax.jit
def kernel(features, targets):
    zsum = jnp.zeros((KP, D), jnp.float32)
    zc = jnp.zeros((128 * 128,), jnp.float32)
    sums, cnt = _sc_segment_sums(features, targets, zsum, zc)
    t3 = targets.reshape(N // 512, 1, 512)
    p = _tc_partial(t3, features)
    fma, mu = _tc_epilogue(sums, cnt.reshape(NW, 128, 128), p)
    return fma, mu.reshape(D)


# speedup vs baseline: 2.4476x; 1.3408x over previous
import jax
import jax.numpy as jnp
from jax import lax
from jax.experimental import pallas as pl
from jax.experimental.pallas import tpu as pltpu
from jax.experimental.pallas import tpu_sc as plsc

N = 320000
D = 128
K = 1000
KP = 1024

NUM_CORES = 2
NUM_SUBCORES = 16
NW = NUM_CORES * NUM_SUBCORES

BT = 1024
M_TC = 80 * BT
SC_ROWS = N - M_TC
ROWS_PER_TILE = SC_ROWS // NW
CNT_ROWS_PER_TILE = N // NW
SCATTER_B = 120
CHUNK = 2 * SCATTER_B
CHUNKS_PER_TILE = ROWS_PER_TILE // CHUNK


def _sc_body(feat_hbm, tgt1_hbm, zsum_hbm, zc_hbm,
             sums_out, cnt_out,
             fbuf0, fbuf1, tb00, tb01, tb10, tb11, tv, cnt_local, acc_sh,
             sem_in0, sem_in1, sem_s0, sem_s1):
    c = lax.axis_index("c")
    s = lax.axis_index("s")
    wid = c * NUM_SUBCORES + s

    fbufs = (fbuf0, fbuf1)
    tbs = ((tb00, tb01), (tb10, tb11))
    sem_in = (sem_in0, sem_in1)
    sem_s = (sem_s0, sem_s1)

    zrows = KP // NUM_SUBCORES
    pltpu.sync_copy(zsum_hbm.at[pl.ds(s * zrows, zrows)],
                    acc_sh.at[pl.ds(s * zrows, zrows)])
    pltpu.sync_copy(zc_hbm, cnt_local)
    pltpu.sync_copy(
        tgt1_hbm.at[pl.ds(wid * CNT_ROWS_PER_TILE, CNT_ROWS_PER_TILE)], tv)
    plsc.subcore_barrier()

    base_f = M_TC + wid * ROWS_PER_TILE

    def start_in(i, b):
        pltpu.async_copy(feat_hbm.at[pl.ds(base_f + i * CHUNK, CHUNK)],
                         fbufs[b], sem_in[b])
        pltpu.async_copy(tgt1_hbm.at[pl.ds(base_f + i * CHUNK, SCATTER_B)],
                         tbs[b][0], sem_in[b])
        pltpu.async_copy(
            tgt1_hbm.at[pl.ds(base_f + i * CHUNK + SCATTER_B, SCATTER_B)],
            tbs[b][1], sem_in[b])

    def wait_in(i, b):
        pltpu.make_async_copy(feat_hbm.at[pl.ds(base_f + i * CHUNK, CHUNK)],
                              fbufs[b], sem_in[b]).wait()
        pltpu.make_async_copy(tgt1_hbm.at[pl.ds(base_f + i * CHUNK, SCATTER_B)],
                              tbs[b][0], sem_in[b]).wait()
        pltpu.make_async_copy(
            tgt1_hbm.at[pl.ds(base_f + i * CHUNK + SCATTER_B, SCATTER_B)],
            tbs[b][1], sem_in[b]).wait()

    def start_scatter(b):
        for h in range(2):
            pltpu.async_copy(fbufs[b].at[pl.ds(h * SCATTER_B, SCATTER_B)],
                             acc_sh.at[tbs[b][h]], sem_s[b], add=True)

    def wait_scatter(b):
        for h in range(2):
            pltpu.make_async_copy(fbufs[b].at[pl.ds(h * SCATTER_B, SCATTER_B)],
                                  acc_sh.at[tbs[b][h]], sem_s[b]).wait()

    start_in(0, 0)

    def pair(j, carry):
        i0 = 2 * j
        wait_in(i0, 0)
        start_scatter(0)

        @pl.when(j > 0)
        def _():
            wait_scatter(1)
        start_in(i0 + 1, 1)
        wait_in(i0 + 1, 1)
        start_scatter(1)

        @pl.when(2 * j + 2 < CHUNKS_PER_TILE)
        def _():
            wait_scatter(0)
            start_in(i0 + 2, 0)
        return carry

    lax.fori_loop(0, CHUNKS_PER_TILE // 2, pair, 0)
    if CHUNKS_PER_TILE % 2:
        wait_in(CHUNKS_PER_TILE - 1, 0)
        start_scatter(0)

    iota16 = lax.iota(jnp.int32, 16)
    ones16 = jnp.ones((16,), jnp.float32)

    def cbody(i, carry):
        t16 = tv[pl.ds(i * 16, 16)]
        idx = ((t16 >> 7) << 11) + (iota16 << 7) + (t16 & 127)
        plsc.addupdate_scatter(cnt_local, [idx], ones16)
        return carry

    lax.fori_loop(0, CNT_ROWS_PER_TILE // 16, cbody, 0)
    pltpu.sync_copy(cnt_local, cnt_out.at[wid])

    wait_scatter(0)
    wait_scatter(1)
    plsc.subcore_barrier()

    @pl.when(s == 0)
    def _():
        pltpu.sync_copy(acc_sh, sums_out.at[c])


_sc_segment_sums = pl.kernel(
    _sc_body,
    out_type=(
        jax.ShapeDtypeStruct((NUM_CORES, KP, D), jnp.float32),
        jax.ShapeDtypeStruct((NW, 128 * 128), jnp.float32),
    ),
    mesh=plsc.VectorSubcoreMesh(core_axis_name="c", subcore_axis_name="s"),
    compiler_params=pltpu.CompilerParams(use_tc_tiling_on_sc=False,
                                         needs_layout_passes=False),
    scratch_types=[
        pltpu.VMEM((CHUNK, D), jnp.float32),
        pltpu.VMEM((CHUNK, D), jnp.float32),
        pltpu.VMEM((SCATTER_B,), jnp.int32),
        pltpu.VMEM((SCATTER_B,), jnp.int32),
        pltpu.VMEM((SCATTER_B,), jnp.int32),
        pltpu.VMEM((SCATTER_B,), jnp.int32),
        pltpu.VMEM((CNT_ROWS_PER_TILE,), jnp.int32),
        pltpu.VMEM((128 * 128,), jnp.float32),
        pltpu.VMEM_SHARED((KP, D), jnp.float32),
        pltpu.SemaphoreType.DMA,
        pltpu.SemaphoreType.DMA,
        pltpu.SemaphoreType.DMA,
        pltpu.SemaphoreType.DMA,
    ],
)


def _tc_partial_body(t_ref, f_ref, p_ref):
    i = pl.program_id(0)

    @pl.when(i == 0)
    def _():
        p_ref[...] = jnp.zeros_like(p_ref)

    fb = f_ref[...].astype(jnp.bfloat16)
    kio = lax.broadcasted_iota(jnp.int32, (KP, 512), 0)
    acc = None
    for h in range(2):
        t = t_ref[h]
        oh_t = (kio == t).astype(jnp.bfloat16)
        part = lax.dot_general(
            oh_t, fb[h * 512:(h + 1) * 512, :],
            (((1,), (0,)), ((), ())),
            preferred_element_type=jnp.float32)
        acc = part if acc is None else acc + part
    p_ref[...] += acc


_tc_partial = pl.pallas_call(
    _tc_partial_body,
    grid=(M_TC // BT,),
    in_specs=[
        pl.BlockSpec((2, 1, 512), lambda i: (i, 0, 0)),
        pl.BlockSpec((BT, D), lambda i: (i, 0)),
    ],
    out_specs=pl.BlockSpec((KP, D), lambda i: (0, 0)),
    out_shape=jax.ShapeDtypeStruct((KP, D), jnp.float32),
)


def _tc_body(sums_ref, cnt_ref, p_ref, fma_ref, mu_ref):
    s = sums_ref[0] + sums_ref[1] + p_ref[...]
    cnt_a = jnp.sum(cnt_ref[...], axis=0)
    b = jnp.sum(cnt_a.reshape(8, 16, 128), axis=1)
    kk = lax.broadcasted_iota(jnp.int32, (KP, 1), 0)
    sel = (lax.broadcasted_iota(jnp.int32, (KP, 8), 1) == (kk >> 7))
    c1 = jnp.dot(sel.astype(jnp.float32), b,
                 preferred_element_type=jnp.float32)
    m_iota = lax.broadcasted_iota(jnp.int32, (KP, 128), 1)
    pick = (m_iota == (kk & 127)).astype(jnp.float32)
    cnt = jnp.sum(c1 * pick, axis=1, keepdims=True)

    mu = jnp.sum(s, axis=0, keepdims=True) / float(N)
    has = cnt > 0.0
    fm = jnp.where(has, s / jnp.where(has, cnt, 1.0) - mu, mu)
    fm_t = fm.T[:, :K]
    norm = jnp.sqrt(jnp.sum(fm_t * fm_t))
    fma_ref[...] = fm_t / norm
    mu_ref[...] = mu


_tc_epilogue = pl.pallas_call(
    _tc_body,
    out_shape=(
        jax.ShapeDtypeStruct((D, K), jnp.float32),
        jax.ShapeDtypeStruct((1, D), jnp.float32),
    ),
)


@jax.jit
def kernel(features, targets):
    zsum = jnp.zeros((KP, D), jnp.float32)
    zc = jnp.zeros((128 * 128,), jnp.float32)
    sums, cnt = _sc_segment_sums(features, targets, zsum, zc)
    t3 = targets.reshape(N // 512, 1, 512)
    p = _tc_partial(t3, features)
    fma, mu = _tc_epilogue(sums, cnt.reshape(NW, 128, 128), p)
    return fma, mu.reshape(D)
